# baseline (speedup 1.0000x reference)
import jax
import jax.numpy as jnp
from jax import lax
from jax.experimental import pallas as pl
from jax.experimental.pallas import tpu as pltpu
from jax.experimental.pallas import tpu_sc as plsc

N = 10000
E = 160000
D = 256
DH = 128
EMB = 64
C_OUT = 2
B = 4096

NC = 2
NS = 16
EPT = 9984
ETAIL = NS * EPT
STG = 2048
RPT = B // NS
AROWS = B + 128
SROWS = AROWS // NS
SELCAP = 10240
NCHUNK = SELCAP // 128


def _sc_body(x_hbm, ei_hbm, nodes_hbm,
             accsel_hbm, xsel_hbm, degsel_hbm,
             nodes_v, slot_v, esd_v, esd2_v, selidx_v, selflat_v, selslot_v,
             rows_v, rows2_v, ones_v, degz_v, slidx_v, degbuf_v,
             acc_sh, deg_sh, sem, sem2, seme, seme2):
    c = lax.axis_index("c")
    s = lax.axis_index("s")
    cdh = pl.multiple_of(c * DH, 128)

    zeros16 = jnp.zeros((16,), jnp.float32)
    ones16 = jnp.ones((16,), jnp.float32)
    iota16 = lax.iota(jnp.int32, 16)

    sc0 = jax.named_scope("p0_stage")
    sc0.__enter__()
    ebase = pl.multiple_of(s * EPT, 128)
    pltpu.async_copy(ei_hbm.at[:, pl.ds(ebase, STG)], esd_v, seme)
    pltpu.sync_copy(nodes_hbm, nodes_v)

    def _zero_rows(i, _):
        for k in range(8):
            rows_v[i, pl.ds(k * 16, 16)] = zeros16
        return 0
    lax.fori_loop(0, 128, _zero_rows, 0)

    def _ini_small(i, _):
        ones_v[pl.ds(i * 16, 16)] = ones16
        degbuf_v[pl.ds(i * 16, 16)] = zeros16
        return 0
    lax.fori_loop(0, 8, _ini_small, 0)

    def _zero_degz(i, _):
        degz_v[pl.ds(i * 16, 16)] = zeros16
        return 0
    lax.fori_loop(0, 272 // 16, _zero_degz, 0)

    r0 = s * SROWS
    pltpu.sync_copy(rows_v, acc_sh.at[pl.ds(r0, 128)])
    pltpu.sync_copy(rows_v, acc_sh.at[pl.ds(r0 + 128, 128)])
    pltpu.sync_copy(rows_v.at[pl.ds(0, 8)], acc_sh.at[pl.ds(r0 + 256, 8)])
    pltpu.sync_copy(degz_v.at[pl.ds(0, SROWS)], deg_sh.at[pl.ds(r0, SROWS)])

    neg1 = jnp.full((16,), -1, jnp.int32)

    def _init_slot(i, _):
        for k in range(5):
            slot_v[pl.ds((i * 5 + k) * 16, 16)] = neg1
        return 0
    lax.fori_loop(0, N // 80, _init_slot, 0)

    def _build_slot(b, _):
        idxv = nodes_v[pl.ds(b * 16, 16)]
        plsc.store_scatter(slot_v, [idxv], b * 16 + iota16)
        return 0
    lax.fori_loop(0, B // 16, _build_slot, 0)

    plsc.subcore_barrier()
    sc0.__exit__(None, None, None)

    sca = jax.named_scope("pA_filter")
    sca.__enter__()

    def _filter_group(buf):
        def _f(i, off):
            srcv = buf[0, pl.ds(i * 16, 16)]
            dstv = buf[1, pl.ds(i * 16, 16)]
            sl = plsc.load_gather(slot_v, [dstv])
            m = sl >= 0
            plsc.store_compressed(selidx_v.at[pl.ds(off, 16)], srcv, mask=m)
            plsc.store_compressed(selflat_v.at[pl.ds(off, 16)], sl, mask=m)
            return off + jnp.sum(jnp.where(m, 1, 0))
        return _f

    off = jnp.int32(0)
    sizes = (STG, STG, STG, STG, EPT - 4 * STG)
    for u in range(5):
        buf, sm = (esd_v, seme) if u % 2 == 0 else (esd2_v, seme2)
        pltpu.make_async_copy(ei_hbm.at[:, pl.ds(0, STG)], buf, sm).wait()
        if u + 1 < 5:
            nbuf, nsm = (esd2_v, seme2) if u % 2 == 0 else (esd_v, seme)
            noff = pl.multiple_of(ebase + (u + 1) * STG, 128)
            pltpu.async_copy(ei_hbm.at[:, pl.ds(noff, STG)], nbuf, nsm)
        off = lax.fori_loop(0, sizes[u] // 16, _filter_group(buf), off)

    toff = pl.multiple_of(ETAIL + (s & 1) * 128, 128)
    pltpu.sync_copy(ei_hbm.at[:, pl.ds(toff, 128)], esd_v.at[:, pl.ds(0, 128)])
    live = s < 2

    def _tfilter(i, off):
        srcv = esd_v[0, pl.ds(i * 16, 16)]
        dstv = esd_v[1, pl.ds(i * 16, 16)]
        sl = plsc.load_gather(slot_v, [dstv])
        m = (sl >= 0) & live
        plsc.store_compressed(selidx_v.at[pl.ds(off, 16)], srcv, mask=m)
        plsc.store_compressed(selflat_v.at[pl.ds(off, 16)], sl, mask=m)
        return off + jnp.sum(jnp.where(m, 1, 0))

    k_total = lax.fori_loop(0, 8, _tfilter, off)
    nchunks = (k_total + 127) // 128

    pad_sl = B + iota16 * 8

    def _pad(g, _):
        selidx_v[pl.ds(k_total + g * 16, 16)] = iota16
        selflat_v[pl.ds(k_total + g * 16, 16)] = pad_sl
        return 0
    lax.fori_loop(0, 8, _pad, 0)

    def _repack(j, _):
        for k in range(8):
            selslot_v[j, pl.ds(k * 16, 16)] = selflat_v[pl.ds(j * 128 + k * 16, 16)]
        return 0
    lax.fori_loop(0, nchunks, _repack, 0)
    sca.__exit__(None, None, None)

    scb = jax.named_scope("pB_agg")
    scb.__enter__()

    def _gather(j, buf, s_):
        pltpu.async_copy(
            x_hbm.at[selidx_v.at[pl.ds(j * 128, 128)], pl.ds(cdh, DH)], buf, s_)

    def _scat(j, buf, s_):
        pltpu.async_copy(buf, acc_sh.at[selslot_v.at[j]], s_, add=True)

        @pl.when(c == 0)
        def _():
            pltpu.sync_copy(ones_v, deg_sh.at[selslot_v.at[j]], add=True)

    def _wait_rows(s_):
        pltpu.make_async_copy(x_hbm.at[pl.ds(0, 128), pl.ds(0, DH)], rows_v, s_).wait()

    @pl.when(nchunks > 0)
    def _():
        _gather(0, rows_v, sem)

    npf = nchunks // 2

    def _pair(jp, _):
        j0 = jp * 2
        j1 = j0 + 1
        _wait_rows(sem)

        @pl.when(jp > 0)
        def _():
            _wait_rows(seme2)
        _gather(j1, rows2_v, sem2)
        _scat(j0, rows_v, seme)
        _wait_rows(sem2)
        _wait_rows(seme)

        @pl.when(j1 + 1 < nchunks)
        def _():
            _gather(j1 + 1, rows_v, sem)
        _scat(j1, rows2_v, seme2)
        return 0
    lax.fori_loop(0, npf, _pair, 0)

    @pl.when(nchunks > 2 * npf)
    def _():
        _wait_rows(sem)
        pltpu.sync_copy(rows_v, acc_sh.at[selslot_v.at[nchunks - 1]], add=True)

        @pl.when(c == 0)
        def _():
            pltpu.sync_copy(ones_v, deg_sh.at[selslot_v.at[nchunks - 1]], add=True)

    @pl.when(npf > 0)
    def _():
        _wait_rows(seme2)

    plsc.subcore_barrier()
    scb.__exit__(None, None, None)

    scc = jax.named_scope("pC_out")
    scc.__enter__()
    for h in range(RPT // 128):
        base = s * RPT + h * 128
        pltpu.async_copy(
            x_hbm.at[nodes_v.at[pl.ds(base, 128)], pl.ds(cdh, DH)], rows2_v, sem2)

        def _mkidx(k, _):
            nb = nodes_v[pl.ds(base + k * 16, 16)]
            slidx_v[pl.ds(k * 16, 16)] = plsc.load_gather(slot_v, [nb])
            return 0
        lax.fori_loop(0, 8, _mkidx, 0)

        pltpu.sync_copy(acc_sh.at[slidx_v], rows_v)
        pltpu.sync_copy(rows_v, accsel_hbm.at[pl.ds(base, 128), pl.ds(cdh, DH)])

        @pl.when(c == 0)
        def _():
            pltpu.sync_copy(deg_sh.at[slidx_v], degbuf_v)
            pltpu.sync_copy(degbuf_v, degsel_hbm.at[pl.ds(base, 128)])

        pltpu.make_async_copy(x_hbm.at[pl.ds(0, 128), pl.ds(0, DH)], rows2_v, sem2).wait()
        pltpu.sync_copy(rows2_v, xsel_hbm.at[pl.ds(base, 128), pl.ds(cdh, DH)])
    scc.__exit__(None, None, None)


_sc_agg = pl.kernel(
    _sc_body,
    out_type=(
        jax.ShapeDtypeStruct((B, D), jnp.float32),
        jax.ShapeDtypeStruct((B, D), jnp.float32),
        jax.ShapeDtypeStruct((B,), jnp.float32),
    ),
    mesh=plsc.VectorSubcoreMesh(core_axis_name="c", subcore_axis_name="s"),
    compiler_params=pltpu.CompilerParams(needs_layout_passes=False),
    scratch_types=[
        pltpu.VMEM((B,), jnp.int32),
        pltpu.VMEM((N,), jnp.int32),
        pltpu.VMEM((2, STG), jnp.int32),
        pltpu.VMEM((2, STG), jnp.int32),
        pltpu.VMEM((SELCAP,), jnp.int32),
        pltpu.VMEM((SELCAP,), jnp.int32),
        pltpu.VMEM((NCHUNK, 128), jnp.int32),
        pltpu.VMEM((128, DH), jnp.float32),
        pltpu.VMEM((128, DH), jnp.float32),
        pltpu.VMEM((128,), jnp.float32),
        pltpu.VMEM((272,), jnp.float32),
        pltpu.VMEM((128,), jnp.int32),
        pltpu.VMEM((128,), jnp.float32),
        pltpu.VMEM_SHARED((AROWS, DH), jnp.float32),
        pltpu.VMEM_SHARED((AROWS,), jnp.float32),
        pltpu.SemaphoreType.DMA,
        pltpu.SemaphoreType.DMA,
        pltpu.SemaphoreType.DMA,
        pltpu.SemaphoreType.DMA,
    ],
)


def _dense_body(xsel_ref, accsel_ref, degsel_ref, wagg_ref, w_ref, out_ref):
    xs = xsel_ref[...]
    ac = accsel_ref[...]
    deg = degsel_ref[...]
    neigh = ac / jnp.maximum(deg, 1.0)
    w1 = wagg_ref[0:D, :]
    w2 = wagg_ref[D:2 * D, :]
    h = jnp.maximum(
        jnp.dot(xs, w1, preferred_element_type=jnp.float32)
        + jnp.dot(neigh, w2, preferred_element_type=jnp.float32), 0.0)
    out_ref[...] = jnp.dot(h, w_ref[...], preferred_element_type=jnp.float32)


def kernel(nodes, x, edge_index, W_agg, weight):
    accsel, xsel, degsel = _sc_agg(x, edge_index, nodes)
    scores = pl.pallas_call(
        _dense_body,
        out_shape=jax.ShapeDtypeStruct((B, C_OUT), jnp.float32),
    )(xsel, accsel, degsel.reshape(B, 1), W_agg, weight)
    return scores


# speedup vs baseline: 1.0704x; 1.0704x over previous
import jax
import jax.numpy as jnp
from jax import lax
from jax.experimental import pallas as pl
from jax.experimental.pallas import tpu as pltpu
from jax.experimental.pallas import tpu_sc as plsc

N = 10000
E = 160000
D = 256
DH = 128
EMB = 64
C_OUT = 2
B = 4096

NC = 2
NS = 16
EPT = 9984
ETAIL = NS * EPT
STG = 2048
RPT = B // NS
AROWS = B + 128
SROWS = AROWS // NS
SELCAP = 10240
NCHUNK = SELCAP // 128


def _sc_body(x_hbm, ei_hbm, nodes_hbm,
             accsel_hbm, xsel_hbm, degsel_hbm,
             nodes_v, slot_v, esd_v, esd2_v, selidx_v, selflat_v, selslot_v,
             rows_v, rows2_v, ones_v, degz_v, slidx_v, degbuf_v,
             acc_sh, deg_sh, sem, sem2, seme, seme2):
    c = lax.axis_index("c")
    s = lax.axis_index("s")
    cdh = pl.multiple_of(c * DH, 128)

    zeros16 = jnp.zeros((16,), jnp.float32)
    ones16 = jnp.ones((16,), jnp.float32)
    iota16 = lax.iota(jnp.int32, 16)

    sc0 = jax.named_scope("p0_stage")
    sc0.__enter__()
    ebase = pl.multiple_of(s * EPT, 128)
    pltpu.async_copy(ei_hbm.at[:, pl.ds(ebase, STG)], esd_v, seme)
    pltpu.sync_copy(nodes_hbm, nodes_v)

    def _zero_rows(i, _):
        for k in range(8):
            rows_v[i, pl.ds(k * 16, 16)] = zeros16
        return 0
    lax.fori_loop(0, 128, _zero_rows, 0)

    def _ini_small(i, _):
        ones_v[pl.ds(i * 16, 16)] = ones16
        degbuf_v[pl.ds(i * 16, 16)] = zeros16
        return 0
    lax.fori_loop(0, 8, _ini_small, 0)

    def _zero_degz(i, _):
        degz_v[pl.ds(i * 16, 16)] = zeros16
        return 0
    lax.fori_loop(0, 272 // 16, _zero_degz, 0)

    r0 = s * SROWS
    pltpu.sync_copy(rows_v, acc_sh.at[pl.ds(r0, 128)])
    pltpu.sync_copy(rows_v, acc_sh.at[pl.ds(r0 + 128, 128)])
    pltpu.sync_copy(rows_v.at[pl.ds(0, 8)], acc_sh.at[pl.ds(r0 + 256, 8)])
    pltpu.sync_copy(degz_v.at[pl.ds(0, SROWS)], deg_sh.at[pl.ds(r0, SROWS)])

    neg1 = jnp.full((16,), -1, jnp.int32)

    def _init_slot(i, _):
        for k in range(5):
            slot_v[pl.ds((i * 5 + k) * 16, 16)] = neg1
        return 0
    lax.fori_loop(0, N // 80, _init_slot, 0)

    def _build_slot(b, _):
        idxv = nodes_v[pl.ds(b * 16, 16)]
        plsc.store_scatter(slot_v, [idxv], b * 16 + iota16)
        return 0
    lax.fori_loop(0, B // 16, _build_slot, 0)

    plsc.subcore_barrier()
    sc0.__exit__(None, None, None)

    sca = jax.named_scope("pA_filter")
    sca.__enter__()

    def _filter_group(buf):
        def _f(i, off):
            srcv = buf[0, pl.ds(i * 16, 16)]
            dstv = buf[1, pl.ds(i * 16, 16)]
            sl = plsc.load_gather(slot_v, [dstv])
            m = sl >= 0
            plsc.store_compressed(selidx_v.at[pl.ds(off, 16)], srcv, mask=m)
            plsc.store_compressed(selflat_v.at[pl.ds(off, 16)], sl, mask=m)
            return off + jnp.sum(jnp.where(m, 1, 0))
        return _f

    off = jnp.int32(0)
    sizes = (STG, STG, STG, STG, EPT - 4 * STG)
    for u in range(5):
        buf, sm = (esd_v, seme) if u % 2 == 0 else (esd2_v, seme2)
        pltpu.make_async_copy(ei_hbm.at[:, pl.ds(0, STG)], buf, sm).wait()
        if u + 1 < 5:
            nbuf, nsm = (esd2_v, seme2) if u % 2 == 0 else (esd_v, seme)
            noff = pl.multiple_of(ebase + (u + 1) * STG, 128)
            pltpu.async_copy(ei_hbm.at[:, pl.ds(noff, STG)], nbuf, nsm)
        off = lax.fori_loop(0, sizes[u] // 16, _filter_group(buf), off)

    toff = pl.multiple_of(ETAIL + (s & 1) * 128, 128)
    pltpu.sync_copy(ei_hbm.at[:, pl.ds(toff, 128)], esd_v.at[:, pl.ds(0, 128)])
    live = s < 2

    def _tfilter(i, off):
        srcv = esd_v[0, pl.ds(i * 16, 16)]
        dstv = esd_v[1, pl.ds(i * 16, 16)]
        sl = plsc.load_gather(slot_v, [dstv])
        m = (sl >= 0) & live
        plsc.store_compressed(selidx_v.at[pl.ds(off, 16)], srcv, mask=m)
        plsc.store_compressed(selflat_v.at[pl.ds(off, 16)], sl, mask=m)
        return off + jnp.sum(jnp.where(m, 1, 0))

    k_total = lax.fori_loop(0, 8, _tfilter, off)
    nchunks = (k_total + 127) // 128

    pad_sl = B + iota16 * 8

    def _pad(g, _):
        selidx_v[pl.ds(k_total + g * 16, 16)] = iota16
        selflat_v[pl.ds(k_total + g * 16, 16)] = pad_sl
        return 0
    lax.fori_loop(0, 8, _pad, 0)

    def _repack(j, _):
        for k in range(8):
            selslot_v[j, pl.ds(k * 16, 16)] = selflat_v[pl.ds(j * 128 + k * 16, 16)]
        return 0
    lax.fori_loop(0, nchunks, _repack, 0)
    sca.__exit__(None, None, None)

    scb = jax.named_scope("pB_agg")
    scb.__enter__()

    def _gather(j, buf, s_):
        pltpu.async_copy(
            x_hbm.at[selidx_v.at[pl.ds(j * 128, 128)], pl.ds(cdh, DH)], buf, s_)

    def _scat(j, buf):
        pltpu.sync_copy(buf, acc_sh.at[selslot_v.at[j]], add=True)

        @pl.when(c == 0)
        def _():
            pltpu.sync_copy(ones_v, deg_sh.at[selslot_v.at[j]], add=True)

    def _wait_rows(s_):
        pltpu.make_async_copy(x_hbm.at[pl.ds(0, 128), pl.ds(0, DH)], rows_v, s_).wait()

    @pl.when(nchunks > 0)
    def _():
        _gather(0, rows_v, sem)

    npairs = (nchunks + 1) // 2

    def _pair(jp, _):
        j0 = jp * 2
        j1 = j0 + 1

        @pl.when(j1 < nchunks)
        def _():
            _gather(j1, rows2_v, sem2)
        _wait_rows(sem)
        _scat(j0, rows_v)

        @pl.when(j1 < nchunks)
        def _():
            @pl.when(j1 + 1 < nchunks)
            def _():
                _gather(j1 + 1, rows_v, sem)
            _wait_rows(sem2)
            _scat(j1, rows2_v)
        return 0
    lax.fori_loop(0, npairs, _pair, 0)

    plsc.subcore_barrier()
    scb.__exit__(None, None, None)

    scc = jax.named_scope("pC_out")
    scc.__enter__()
    for h in range(RPT // 128):
        base = s * RPT + h * 128
        pltpu.async_copy(
            x_hbm.at[nodes_v.at[pl.ds(base, 128)], pl.ds(cdh, DH)], rows2_v, sem2)

        def _mkidx(k, _):
            nb = nodes_v[pl.ds(base + k * 16, 16)]
            slidx_v[pl.ds(k * 16, 16)] = plsc.load_gather(slot_v, [nb])
            return 0
        lax.fori_loop(0, 8, _mkidx, 0)

        pltpu.sync_copy(acc_sh.at[slidx_v], rows_v)
        pltpu.sync_copy(rows_v, accsel_hbm.at[pl.ds(base, 128), pl.ds(cdh, DH)])

        @pl.when(c == 0)
        def _():
            pltpu.sync_copy(deg_sh.at[slidx_v], degbuf_v)
            pltpu.sync_copy(degbuf_v, degsel_hbm.at[pl.ds(base, 128)])

        pltpu.make_async_copy(x_hbm.at[pl.ds(0, 128), pl.ds(0, DH)], rows2_v, sem2).wait()
        pltpu.sync_copy(rows2_v, xsel_hbm.at[pl.ds(base, 128), pl.ds(cdh, DH)])
    scc.__exit__(None, None, None)


_sc_agg = pl.kernel(
    _sc_body,
    out_type=(
        jax.ShapeDtypeStruct((B, D), jnp.float32),
        jax.ShapeDtypeStruct((B, D), jnp.float32),
        jax.ShapeDtypeStruct((B,), jnp.float32),
    ),
    mesh=plsc.VectorSubcoreMesh(core_axis_name="c", subcore_axis_name="s"),
    compiler_params=pltpu.CompilerParams(needs_layout_passes=False),
    scratch_types=[
        pltpu.VMEM((B,), jnp.int32),
        pltpu.VMEM((N,), jnp.int32),
        pltpu.VMEM((2, STG), jnp.int32),
        pltpu.VMEM((2, STG), jnp.int32),
        pltpu.VMEM((SELCAP,), jnp.int32),
        pltpu.VMEM((SELCAP,), jnp.int32),
        pltpu.VMEM((NCHUNK, 128), jnp.int32),
        pltpu.VMEM((128, DH), jnp.float32),
        pltpu.VMEM((128, DH), jnp.float32),
        pltpu.VMEM((128,), jnp.float32),
        pltpu.VMEM((272,), jnp.float32),
        pltpu.VMEM((128,), jnp.int32),
        pltpu.VMEM((128,), jnp.float32),
        pltpu.VMEM_SHARED((AROWS, DH), jnp.float32),
        pltpu.VMEM_SHARED((AROWS,), jnp.float32),
        pltpu.SemaphoreType.DMA,
        pltpu.SemaphoreType.DMA,
        pltpu.SemaphoreType.DMA,
        pltpu.SemaphoreType.DMA,
    ],
)


_DN = (((1,), (1,)), ((), ()))


def _dense_body(xsel_ref, accsel_ref, degsel_ref, waggT_ref, wT_ref, out_ref):
    xs = xsel_ref[...]
    ac = accsel_ref[...]
    deg = degsel_ref[...]
    neigh = ac / jnp.maximum(deg, 1.0)
    w1t = waggT_ref[:, 0:D]
    w2t = waggT_ref[:, D:2 * D]
    h = jnp.maximum(
        lax.dot_general(xs, w1t, _DN, preferred_element_type=jnp.float32)
        + lax.dot_general(neigh, w2t, _DN, preferred_element_type=jnp.float32),
        0.0)
    out_ref[...] = lax.dot_general(
        wT_ref[...], h, _DN, preferred_element_type=jnp.float32)


def kernel(nodes, x, edge_index, W_agg, weight):
    accsel, xsel, degsel = _sc_agg(x, edge_index, nodes)
    scores_t = pl.pallas_call(
        _dense_body,
        out_shape=jax.ShapeDtypeStruct((C_OUT, B), jnp.float32),
    )(xsel, accsel, degsel.reshape(B, 1), W_agg.T, weight.T)
    return scores_t.T


# speedup vs baseline: 1.1086x; 1.0357x over previous
import jax
import jax.numpy as jnp
from jax import lax
from jax.experimental import pallas as pl
from jax.experimental.pallas import tpu as pltpu
from jax.experimental.pallas import tpu_sc as plsc

N = 10000
E = 160000
D = 256
DH = 128
EMB = 64
C_OUT = 2
B = 4096

NC = 2
NS = 16
EPT = 9984
ETAIL = NS * EPT
STG = 2048
RPT = B // NS
AROWS = B + 128
SROWS = AROWS // NS
SELCAP = 10240
NCHUNK = SELCAP // 128


def _sc_body(x_hbm, ei_hbm, nodes_hbm,
             accsel_hbm, xsel_hbm, degsel_hbm,
             nodes_v, slot_v, esd_v, esd2_v, selidx_v, selflat_v, selslot_v,
             rows_v, rows2_v, ones_v, degz_v, slidx_v, degbuf_v,
             acc_sh, deg_sh, sem, sem2, seme, seme2):
    c = lax.axis_index("c")
    s = lax.axis_index("s")
    cdh = pl.multiple_of(c * DH, 128)

    zeros16 = jnp.zeros((16,), jnp.float32)
    ones16 = jnp.ones((16,), jnp.float32)
    iota16 = lax.iota(jnp.int32, 16)

    sc0 = jax.named_scope("p0_stage")
    sc0.__enter__()
    ebase = pl.multiple_of(s * EPT, 128)
    pltpu.async_copy(ei_hbm.at[:, pl.ds(ebase, STG)], esd_v, seme)
    pltpu.sync_copy(nodes_hbm, nodes_v)

    def _zero_rows(i, _):
        for k in range(8):
            rows_v[i, pl.ds(k * 16, 16)] = zeros16
        return 0
    lax.fori_loop(0, 128, _zero_rows, 0)

    def _ini_small(i, _):
        ones_v[pl.ds(i * 16, 16)] = ones16
        degbuf_v[pl.ds(i * 16, 16)] = zeros16
        return 0
    lax.fori_loop(0, 8, _ini_small, 0)

    def _zero_degz(i, _):
        degz_v[pl.ds(i * 16, 16)] = zeros16
        return 0
    lax.fori_loop(0, 272 // 16, _zero_degz, 0)

    r0 = s * SROWS
    pltpu.sync_copy(rows_v, acc_sh.at[pl.ds(r0, 128)])
    pltpu.sync_copy(rows_v, acc_sh.at[pl.ds(r0 + 128, 128)])
    pltpu.sync_copy(rows_v.at[pl.ds(0, 8)], acc_sh.at[pl.ds(r0 + 256, 8)])
    pltpu.sync_copy(degz_v.at[pl.ds(0, SROWS)], deg_sh.at[pl.ds(r0, SROWS)])

    neg1 = jnp.full((16,), -1, jnp.int32)

    def _init_slot(i, _):
        for k in range(5):
            slot_v[pl.ds((i * 5 + k) * 16, 16)] = neg1
        return 0
    lax.fori_loop(0, N // 80, _init_slot, 0)

    def _build_slot(b, _):
        idxv = nodes_v[pl.ds(b * 16, 16)]
        plsc.store_scatter(slot_v, [idxv], b * 16 + iota16)
        return 0
    lax.fori_loop(0, B // 16, _build_slot, 0)

    plsc.subcore_barrier()
    sc0.__exit__(None, None, None)

    sca = jax.named_scope("pA_filter")
    sca.__enter__()

    def _filter_group(buf):
        def _f(i, off):
            srcv = buf[0, pl.ds(i * 16, 16)]
            dstv = buf[1, pl.ds(i * 16, 16)]
            sl = plsc.load_gather(slot_v, [dstv])
            m = sl >= 0
            plsc.store_compressed(selidx_v.at[pl.ds(off, 16)], srcv, mask=m)
            plsc.store_compressed(selflat_v.at[pl.ds(off, 16)], sl, mask=m)
            return off + jnp.sum(jnp.where(m, 1, 0))
        return _f

    off = jnp.int32(0)
    sizes = (STG, STG, STG, STG, EPT - 4 * STG)
    for u in range(5):
        buf, sm = (esd_v, seme) if u % 2 == 0 else (esd2_v, seme2)
        pltpu.make_async_copy(ei_hbm.at[:, pl.ds(0, STG)], buf, sm).wait()
        if u + 1 < 5:
            nbuf, nsm = (esd2_v, seme2) if u % 2 == 0 else (esd_v, seme)
            noff = pl.multiple_of(ebase + (u + 1) * STG, 128)
            pltpu.async_copy(ei_hbm.at[:, pl.ds(noff, STG)], nbuf, nsm)
        off = lax.fori_loop(0, sizes[u] // 16, _filter_group(buf), off)

    toff = pl.multiple_of(ETAIL + (s & 1) * 128, 128)
    pltpu.sync_copy(ei_hbm.at[:, pl.ds(toff, 128)], esd_v.at[:, pl.ds(0, 128)])
    live = s < 2

    def _tfilter(i, off):
        srcv = esd_v[0, pl.ds(i * 16, 16)]
        dstv = esd_v[1, pl.ds(i * 16, 16)]
        sl = plsc.load_gather(slot_v, [dstv])
        m = (sl >= 0) & live
        plsc.store_compressed(selidx_v.at[pl.ds(off, 16)], srcv, mask=m)
        plsc.store_compressed(selflat_v.at[pl.ds(off, 16)], sl, mask=m)
        return off + jnp.sum(jnp.where(m, 1, 0))

    k_total = lax.fori_loop(0, 8, _tfilter, off)
    nchunks = (k_total + 127) // 128

    pad_sl = B + iota16 * 8

    def _pad(g, _):
        selidx_v[pl.ds(k_total + g * 16, 16)] = iota16
        selflat_v[pl.ds(k_total + g * 16, 16)] = pad_sl
        return 0
    lax.fori_loop(0, 8, _pad, 0)

    def _repack(j, _):
        for k in range(8):
            selslot_v[j, pl.ds(k * 16, 16)] = selflat_v[pl.ds(j * 128 + k * 16, 16)]
        return 0
    lax.fori_loop(0, nchunks, _repack, 0)
    sca.__exit__(None, None, None)

    scb = jax.named_scope("pB_agg")
    scb.__enter__()

    def _gather(j, buf, s_):
        pltpu.async_copy(
            x_hbm.at[selidx_v.at[pl.ds(j * 128, 128)], pl.ds(cdh, DH)], buf, s_)

    def _scat(j, buf):
        pltpu.sync_copy(buf, acc_sh.at[selslot_v.at[j]], add=True)

        @pl.when(c == 0)
        def _():
            pltpu.sync_copy(ones_v, deg_sh.at[selslot_v.at[j]], add=True)

    def _wait_rows(s_):
        pltpu.make_async_copy(x_hbm.at[pl.ds(0, 128), pl.ds(0, DH)], rows_v, s_).wait()

    @pl.when(nchunks > 0)
    def _():
        _gather(0, rows_v, sem)

    npairs = (nchunks + 1) // 2

    def _pair(jp, _):
        j0 = jp * 2
        j1 = j0 + 1

        @pl.when(j1 < nchunks)
        def _():
            _gather(j1, rows2_v, sem2)
        _wait_rows(sem)
        _scat(j0, rows_v)

        @pl.when(j1 < nchunks)
        def _():
            @pl.when(j1 + 1 < nchunks)
            def _():
                _gather(j1 + 1, rows_v, sem)
            _wait_rows(sem2)
            _scat(j1, rows2_v)
        return 0
    lax.fori_loop(0, npairs, _pair, 0)

    plsc.subcore_barrier()
    scb.__exit__(None, None, None)

    scc = jax.named_scope("pC_out")
    scc.__enter__()
    for h in range(RPT // 128):
        base = s * RPT + h * 128
        pltpu.async_copy(
            x_hbm.at[nodes_v.at[pl.ds(base, 128)], pl.ds(cdh, DH)], rows2_v, sem2)

        def _mkidx(k, _):
            nb = nodes_v[pl.ds(base + k * 16, 16)]
            slidx_v[pl.ds(k * 16, 16)] = plsc.load_gather(slot_v, [nb])
            return 0
        lax.fori_loop(0, 8, _mkidx, 0)

        pltpu.sync_copy(acc_sh.at[slidx_v], rows_v)
        pltpu.sync_copy(rows_v, accsel_hbm.at[pl.ds(base, 128), pl.ds(cdh, DH)])

        @pl.when(c == 0)
        def _():
            pltpu.sync_copy(deg_sh.at[slidx_v], degbuf_v)
            pltpu.sync_copy(degbuf_v, degsel_hbm.at[pl.ds(base, 128)])

        pltpu.make_async_copy(x_hbm.at[pl.ds(0, 128), pl.ds(0, DH)], rows2_v, sem2).wait()
        pltpu.sync_copy(rows2_v, xsel_hbm.at[pl.ds(base, 128), pl.ds(cdh, DH)])
    scc.__exit__(None, None, None)


_sc_agg = pl.kernel(
    _sc_body,
    out_type=(
        jax.ShapeDtypeStruct((B, D), jnp.float32),
        jax.ShapeDtypeStruct((B, D), jnp.float32),
        jax.ShapeDtypeStruct((B,), jnp.float32),
    ),
    mesh=plsc.VectorSubcoreMesh(core_axis_name="c", subcore_axis_name="s"),
    compiler_params=pltpu.CompilerParams(needs_layout_passes=False),
    scratch_types=[
        pltpu.VMEM((B,), jnp.int32),
        pltpu.VMEM((N,), jnp.int32),
        pltpu.VMEM((2, STG), jnp.int32),
        pltpu.VMEM((2, STG), jnp.int32),
        pltpu.VMEM((SELCAP,), jnp.int32),
        pltpu.VMEM((SELCAP,), jnp.int32),
        pltpu.VMEM((NCHUNK, 128), jnp.int32),
        pltpu.VMEM((128, DH), jnp.float32),
        pltpu.VMEM((128, DH), jnp.float32),
        pltpu.VMEM((128,), jnp.float32),
        pltpu.VMEM((272,), jnp.float32),
        pltpu.VMEM((128,), jnp.int32),
        pltpu.VMEM((128,), jnp.float32),
        pltpu.VMEM_SHARED((AROWS, DH), jnp.float32),
        pltpu.VMEM_SHARED((AROWS,), jnp.float32),
        pltpu.SemaphoreType.DMA,
        pltpu.SemaphoreType.DMA,
        pltpu.SemaphoreType.DMA,
        pltpu.SemaphoreType.DMA,
    ],
)


_DN = (((1,), (1,)), ((), ()))


def _dense_body(xsel_ref, accsel_ref, degsel_ref, waggT_ref, wT_ref, out_ref):
    xs = xsel_ref[...]
    ac = accsel_ref[...]
    deg = jnp.reshape(degsel_ref[...], (B, 1))
    neigh = ac / jnp.maximum(deg, 1.0)
    w1t = waggT_ref[:, 0:D]
    w2t = waggT_ref[:, D:2 * D]
    h = jnp.maximum(
        lax.dot_general(xs, w1t, _DN, preferred_element_type=jnp.float32)
        + lax.dot_general(neigh, w2t, _DN, preferred_element_type=jnp.float32),
        0.0)
    out_ref[...] = lax.dot_general(
        wT_ref[...], h, _DN, preferred_element_type=jnp.float32)


def kernel(nodes, x, edge_index, W_agg, weight):
    accsel, xsel, degsel = _sc_agg(x, edge_index, nodes)
    scores_t = pl.pallas_call(
        _dense_body,
        out_shape=jax.ShapeDtypeStruct((C_OUT, B), jnp.float32),
    )(xsel, accsel, degsel, W_agg.T, weight.T)
    return scores_t.T
